# baseline (speedup 1.0000x reference)
import jax
import jax.numpy as jnp
from jax import lax
from jax.experimental import pallas as pl
from jax.experimental.pallas import tpu as pltpu
from jax.experimental.pallas import tpu_sc as plsc

D = 64
NB = 16
SEQ = 2048
NW = 32
SPW = SEQ // NW
B_TOT = NB * SEQ


def _scalar(vec, i):
    return lax.squeeze(lax.slice(vec, (i,), (i + 1,)), (0,))


def _body(xr_hbm, tok_hbm, pos_hbm, out_hbm, idx_v, pos_v, stage_v, sem):
    c = lax.axis_index("c")
    s = lax.axis_index("s")
    wid = s * 2 + c
    s0 = wid * SPW

    pltpu.sync_copy(xr_hbm.at[wid], idx_v)
    pltpu.sync_copy(pos_hbm.at[pl.ds(s0, SPW)], pos_v)

    def block(b, carry):
        handles = []
        for g in range(SPW // 16):
            v16 = idx_v[b, pl.ds(g * 16, 16)]
            for i in range(16):
                k = g * 16 + i
                v = _scalar(v16, i)
                handles.append(pltpu.async_copy(
                    tok_hbm.at[pl.ds(v, 1)], stage_v.at[pl.ds(k, 1)], sem))
        for h in handles:
            h.wait()

        for k in range(SPW):
            for t in range(D // 16):
                stage_v[k, pl.ds(t * 16, 16)] = (
                    stage_v[k, pl.ds(t * 16, 16)]
                    + pos_v[k, pl.ds(t * 16, 16)]
                )

        pltpu.sync_copy(stage_v, out_hbm.at[pl.ds(b * SEQ + s0, SPW)])
        return carry

    lax.fori_loop(0, NB, block, 0)


@jax.jit
def _embed(xr, tok, pos_table):
    mesh = plsc.VectorSubcoreMesh(core_axis_name="c", subcore_axis_name="s")
    return pl.kernel(
        _body,
        out_type=jax.ShapeDtypeStruct((B_TOT, D), jnp.float32),
        mesh=mesh,
        scratch_types=[
            pltpu.VMEM((NB, SPW), jnp.int32),
            pltpu.VMEM((SPW, D), jnp.float32),
            pltpu.VMEM((SPW, D), jnp.float32),
            pltpu.SemaphoreType.DMA,
        ],
        compiler_params=pltpu.CompilerParams(use_tc_tiling_on_sc=False),
    )(xr, tok, pos_table)


def kernel(x, token_table, pos_table):
    xr = x.astype(jnp.int32).reshape(NB, NW, SPW).transpose(1, 0, 2)
    out = _embed(xr, token_table, pos_table)
    return out.reshape(x.shape[0], x.shape[1], D)


# speedup vs baseline: 1.6499x; 1.6499x over previous
import jax
import jax.numpy as jnp
from jax import lax
from jax.experimental import pallas as pl
from jax.experimental.pallas import tpu as pltpu
from jax.experimental.pallas import tpu_sc as plsc

D = 64
NB = 16
SEQ = 2048
NW = 32
SPW = SEQ // NW
B_TOT = NB * SEQ


def _scalar(vec, i):
    return lax.squeeze(lax.slice(vec, (i,), (i + 1,)), (0,))


def _body(xr_hbm, tok_hbm, pos_hbm, out_hbm, idx_v, pos_v, stage_v, sem):
    c = lax.axis_index("c")
    s = lax.axis_index("s")
    wid = s * 2 + c
    s0 = wid * SPW

    pltpu.sync_copy(xr_hbm.at[wid], idx_v)
    pltpu.sync_copy(pos_hbm.at[pl.ds(s0, SPW)], pos_v)

    def block(b, carry):
        handles = []
        for g in range(SPW // 16):
            v16 = idx_v[b, pl.ds(g * 16, 16)]
            for i in range(16):
                k = g * 16 + i
                v = _scalar(v16, i)
                handles.append(pltpu.async_copy(
                    tok_hbm.at[pl.ds(v, 1)], stage_v.at[pl.ds(k, 1)], sem))
        for h in handles:
            h.wait()

        for k in range(SPW):
            for t in range(D // 16):
                stage_v[k, pl.ds(t * 16, 16)] = (
                    stage_v[k, pl.ds(t * 16, 16)]
                    + pos_v[k, pl.ds(t * 16, 16)]
                )

        pltpu.sync_copy(stage_v, out_hbm.at[pl.ds(b * SEQ + s0, SPW)])
        return carry

    lax.fori_loop(0, NB, block, 0)


@jax.jit
def _embed(xr, tok, pos_table):
    mesh = plsc.VectorSubcoreMesh(core_axis_name="c", subcore_axis_name="s")
    return pl.kernel(
        _body,
        out_type=jax.ShapeDtypeStruct((B_TOT, D), jnp.float32),
        mesh=mesh,
        scratch_types=[
            pltpu.VMEM((NB, SPW), jnp.int32),
            pltpu.VMEM((SPW, D), jnp.float32),
            pltpu.VMEM((SPW, D), jnp.float32),
            pltpu.SemaphoreType.DMA,
        ],
        compiler_params=pltpu.CompilerParams(needs_layout_passes=False),
    )(xr, tok, pos_table)


def kernel(x, token_table, pos_table):
    xr = x.astype(jnp.int32).reshape(NB, NW, SPW).transpose(1, 0, 2)
    out = _embed(xr, token_table, pos_table)
    return out.reshape(x.shape[0], x.shape[1], D)
